# baseline (speedup 1.0000x reference)
import functools

import jax
import jax.numpy as jnp
from jax.experimental import pallas as pl
from jax.experimental.pallas import tpu as pltpu

_EPS = 1e-5


def _accum_partial(q, sum_acc, sq_acc):
    for t in range(0, q.shape[1], 128):
        sl = q[:, t:t + 128]
        sum_acc[...] += sl
        sq_acc[...] += sl * sl


def _finalize_affine(sum_in, sq_in, gamma, beta, b, m, scale_scr, shift_scr):
    qsum = jnp.sum(sum_in, axis=1, keepdims=True)
    qsq = jnp.sum(sq_in, axis=1, keepdims=True)
    psum = qsum + m * b
    psq = qsq + 2.0 * b * qsum + m * b * b
    mean = psum / m
    var = jnp.maximum(psq / m - mean * mean, 0.0)
    scale = gamma * jax.lax.rsqrt(var + _EPS)
    scale_scr[...] = scale
    shift_scr[...] = beta - mean * scale + scale * b


def _head_body(x_ref, w1_ref, b1_ref, wh0_ref, q_ref, sum_ref, sq_ref,
               sum_acc, sq_acc, *, group_b, steps):
    i = pl.program_id(0)

    @pl.when(i == 0)
    def _():
        sum_acc[...] = jnp.zeros_like(sum_acc)
        sq_acc[...] = jnp.zeros_like(sq_acc)

    hs = [jnp.maximum(
        jnp.dot(w1_ref[...], x_ref[g], preferred_element_type=jnp.float32)
        + b1_ref[...], 0.0) for g in range(group_b)]
    hcat = jnp.concatenate(hs, axis=1)
    q = jnp.dot(wh0_ref[...], hcat, preferred_element_type=jnp.float32)
    q_ref[...] = q.astype(q_ref.dtype)
    _accum_partial(q, sum_acc, sq_acc)

    @pl.when(i == steps - 1)
    def _():
        sum_ref[...] = sum_acc[...]
        sq_ref[...] = sq_acc[...]


def _mid_body(q_in_ref, sum_in_ref, sq_in_ref, gamma_ref, beta_ref, bprev_ref,
              wh_ref, q_ref, sum_ref, sq_ref,
              sum_acc, sq_acc, scale_scr, shift_scr, *, steps, m):
    i = pl.program_id(0)

    @pl.when(i == 0)
    def _():
        _finalize_affine(sum_in_ref[...], sq_in_ref[...], gamma_ref[...],
                         beta_ref[...], bprev_ref[...], m,
                         scale_scr, shift_scr)
        sum_acc[...] = jnp.zeros_like(sum_acc)
        sq_acc[...] = jnp.zeros_like(sq_acc)

    h = jnp.maximum(
        q_in_ref[...].astype(jnp.float32) * scale_scr[...] + shift_scr[...],
        0.0)
    q = jnp.dot(wh_ref[...], h, preferred_element_type=jnp.float32)
    q_ref[...] = q.astype(q_ref.dtype)
    _accum_partial(q, sum_acc, sq_acc)

    @pl.when(i == steps - 1)
    def _():
        sum_ref[...] = sum_acc[...]
        sq_ref[...] = sq_acc[...]


def _tail_body(q_in_ref, sum_in_ref, sq_in_ref, gamma_ref, beta_ref, bprev_ref,
               wl_ref, bl_ref, o_ref, scale_scr, shift_scr, *, group_b, l, m):
    i = pl.program_id(0)

    @pl.when(i == 0)
    def _():
        _finalize_affine(sum_in_ref[...], sq_in_ref[...], gamma_ref[...],
                         beta_ref[...], bprev_ref[...], m,
                         scale_scr, shift_scr)

    h = jnp.maximum(
        q_in_ref[...].astype(jnp.float32) * scale_scr[...] + shift_scr[...],
        0.0)
    out = jnp.dot(wl_ref[...], h, preferred_element_type=jnp.float32) + bl_ref[...]
    for g in range(group_b):
        o_ref[g] = out[:, g * l:(g + 1) * l].astype(o_ref.dtype)


def _const(a):
    return pl.BlockSpec(a.shape, lambda *_: (0,) * a.ndim)


def kernel(x, w1, b1, wh, bh, gamma, beta, wl, bl):
    n, cin, l = x.shape
    d = w1.shape[0]
    cout = wl.shape[0]
    n_hidden = wh.shape[0]
    m = float(n * l)

    group_b = next(g for g in (4, 2, 1) if n % g == 0)
    steps = n // group_b
    wl_lanes = group_b * l

    x_spec = pl.BlockSpec((group_b, cin, l), lambda i: (i, 0, 0))
    q_spec = pl.BlockSpec((None, d, wl_lanes), lambda i: (i, 0, 0))
    acc_spec = pl.BlockSpec((d, 128), lambda i: (0, 0))
    acc_shape = jax.ShapeDtypeStruct((d, 128), jnp.float32)
    acc_scratch = [pltpu.VMEM((d, 128), jnp.float32),
                   pltpu.VMEM((d, 128), jnp.float32)]
    affine_scratch = [pltpu.VMEM((d, 1), jnp.float32),
                      pltpu.VMEM((d, 1), jnp.float32)]
    params = pltpu.CompilerParams(
        dimension_semantics=("arbitrary",),
        vmem_limit_bytes=60 * 1024 * 1024)

    q, psum, psq = pl.pallas_call(
        functools.partial(_head_body, group_b=group_b, steps=steps),
        grid=(steps,),
        in_specs=[x_spec, _const(w1), _const(b1), _const(wh[0])],
        out_specs=(q_spec, acc_spec, acc_spec),
        out_shape=(jax.ShapeDtypeStruct((steps, d, wl_lanes), jnp.bfloat16),
                   acc_shape, acc_shape),
        scratch_shapes=acc_scratch,
        compiler_params=params,
    )(x, w1, b1, wh[0])

    for j in range(1, n_hidden):
        q, psum, psq = pl.pallas_call(
            functools.partial(_mid_body, steps=steps, m=m),
            grid=(steps,),
            in_specs=[q_spec, _const(psum), _const(psq), _const(gamma[j - 1]),
                      _const(beta[j - 1]), _const(bh[j - 1]), _const(wh[j])],
            out_specs=(q_spec, acc_spec, acc_spec),
            out_shape=(jax.ShapeDtypeStruct((steps, d, wl_lanes), jnp.bfloat16),
                       acc_shape, acc_shape),
            scratch_shapes=acc_scratch + affine_scratch,
            compiler_params=params,
        )(q, psum, psq, gamma[j - 1], beta[j - 1], bh[j - 1], wh[j])

    return pl.pallas_call(
        functools.partial(_tail_body, group_b=group_b, l=l, m=m),
        grid=(steps,),
        in_specs=[q_spec, _const(psum), _const(psq), _const(gamma[-1]),
                  _const(beta[-1]), _const(bh[-1]), _const(wl), _const(bl)],
        out_specs=pl.BlockSpec((group_b, cout, l), lambda i: (i, 0, 0)),
        out_shape=jax.ShapeDtypeStruct((n, cout, l), x.dtype),
        scratch_shapes=affine_scratch,
        compiler_params=params,
    )(q, psum, psq, gamma[-1], beta[-1], bh[-1], wl, bl)


# speedup vs baseline: 1.1746x; 1.1746x over previous
import functools

import jax
import jax.numpy as jnp
from jax.experimental import pallas as pl
from jax.experimental.pallas import tpu as pltpu

_EPS = 1e-5


def _accum_partial(q, sum_acc, sq_acc):
    for t in range(0, q.shape[1], 128):
        sl = q[:, t:t + 128]
        sum_acc[...] += sl
        sq_acc[...] += sl * sl


def _fused_body(x_ref, w1_ref, b1_ref, wh_ref, bh_ref, gamma_ref, beta_ref,
                wl_ref, bl_ref, o_ref,
                p_scr, sum_acc, sq_acc, scale_scr, shift_scr,
                *, group_b, n_stages, l, m):
    s = pl.program_id(0)
    i = pl.program_id(1)
    inv_m = 1.0 / m

    @pl.when(i == 0)
    def _boundary():
        @pl.when(s > 0)
        def _():
            b = bh_ref[s - 1]
            qsum = jnp.sum(sum_acc[...], axis=1, keepdims=True)
            qsq = jnp.sum(sq_acc[...], axis=1, keepdims=True)
            psum = qsum + m * b
            psq = qsq + 2.0 * b * qsum + m * b * b
            mean = psum * inv_m
            var = jnp.maximum(psq * inv_m - mean * mean, 0.0)
            scale = gamma_ref[s - 1] * jax.lax.rsqrt(var + _EPS)
            scale_scr[...] = scale
            shift_scr[...] = beta_ref[s - 1] - mean * scale + scale * b
        sum_acc[...] = jnp.zeros_like(sum_acc)
        sq_acc[...] = jnp.zeros_like(sq_acc)

    @pl.when(s == 0)
    def _phase_in():
        hs = [jnp.maximum(
            jnp.dot(w1_ref[...], x_ref[g], preferred_element_type=jnp.float32)
            + b1_ref[...], 0.0) for g in range(group_b)]
        hcat = jnp.concatenate(hs, axis=1)
        q = jnp.dot(wh_ref[0], hcat, preferred_element_type=jnp.float32)
        p_scr[i] = q
        _accum_partial(q, sum_acc, sq_acc)

    if n_stages > 2:
        @pl.when(jnp.logical_and(s > 0, s < n_stages - 1))
        def _phase_mid():
            h = jnp.maximum(p_scr[i] * scale_scr[...] + shift_scr[...], 0.0)
            q = jnp.dot(wh_ref[s], h, preferred_element_type=jnp.float32)
            p_scr[i] = q
            _accum_partial(q, sum_acc, sq_acc)

    @pl.when(s == n_stages - 1)
    def _phase_out():
        h = jnp.maximum(p_scr[i] * scale_scr[...] + shift_scr[...], 0.0)
        out = jnp.dot(wl_ref[...], h, preferred_element_type=jnp.float32) + bl_ref[...]
        for g in range(group_b):
            o_ref[g] = out[:, g * l:(g + 1) * l].astype(o_ref.dtype)


def kernel(x, w1, b1, wh, bh, gamma, beta, wl, bl):
    n, cin, l = x.shape
    d = w1.shape[0]
    cout = wl.shape[0]
    n_hidden = wh.shape[0]
    n_stages = n_hidden + 1
    last = n_stages - 1

    group_b = next(g for g in (4, 2, 1) if n % g == 0)
    steps = n // group_b

    body = functools.partial(_fused_body, group_b=group_b, n_stages=n_stages,
                             l=l, m=float(n * l))

    x_spec = pl.BlockSpec((group_b, cin, l), lambda s, i: (i, 0, 0))
    o_spec = pl.BlockSpec(
        (group_b, cout, l),
        lambda s, i: (jnp.where(s == last, i, 0), 0, 0))

    def const(a):
        return pl.BlockSpec(a.shape, lambda *_: (0,) * a.ndim)

    return pl.pallas_call(
        body,
        grid=(n_stages, steps),
        in_specs=[x_spec, const(w1), const(b1), const(wh), const(bh),
                  const(gamma), const(beta), const(wl), const(bl)],
        out_specs=o_spec,
        out_shape=jax.ShapeDtypeStruct((n, cout, l), x.dtype),
        scratch_shapes=[
            pltpu.VMEM((steps, d, group_b * l), jnp.float32),
            pltpu.VMEM((d, 128), jnp.float32),
            pltpu.VMEM((d, 128), jnp.float32),
            pltpu.VMEM((d, 1), jnp.float32),
            pltpu.VMEM((d, 1), jnp.float32),
        ],
        compiler_params=pltpu.CompilerParams(
            dimension_semantics=("arbitrary", "arbitrary"),
            vmem_limit_bytes=60 * 1024 * 1024),
    )(x, w1, b1, wh, bh, gamma, beta, wl, bl)


# speedup vs baseline: 1.4787x; 1.2589x over previous
import functools

import jax
import jax.numpy as jnp
from jax.experimental import pallas as pl
from jax.experimental.pallas import tpu as pltpu

_EPS = 1e-5


def _accum_partial(q, sum_acc, sq_acc):
    for t in range(0, q.shape[1], 128):
        sl = q[:, t:t + 128]
        sum_acc[...] += sl
        sq_acc[...] += sl * sl


def _fused_body(x_ref, w1_ref, b1_ref, wh_ref, bh_ref, gamma_ref, beta_ref,
                wl_ref, bl_ref, o_ref,
                p_scr, sum_acc, sq_acc, scale_scr, shift_scr,
                *, group_b, n_stages, l, m):
    s = pl.program_id(0)
    i = pl.program_id(1)
    inv_m = 1.0 / m

    @pl.when(i == 0)
    def _boundary():
        @pl.when(s > 0)
        def _():
            b = bh_ref[s - 1]
            qsum = jnp.sum(sum_acc[...], axis=1, keepdims=True)
            qsq = jnp.sum(sq_acc[...], axis=1, keepdims=True)
            psum = qsum + m * b
            psq = qsq + 2.0 * b * qsum + m * b * b
            mean = psum * inv_m
            var = jnp.maximum(psq * inv_m - mean * mean, 0.0)
            scale = gamma_ref[s - 1] * jax.lax.rsqrt(var + _EPS)
            scale_scr[...] = scale
            shift_scr[...] = beta_ref[s - 1] - mean * scale + scale * b
        sum_acc[...] = jnp.zeros_like(sum_acc)
        sq_acc[...] = jnp.zeros_like(sq_acc)

    @pl.when(s == 0)
    def _phase_in():
        hs = [jnp.maximum(
            jnp.dot(w1_ref[...], x_ref[g], preferred_element_type=jnp.float32)
            + b1_ref[...], 0.0) for g in range(group_b)]
        hcat = jnp.concatenate(hs, axis=1)
        q = jnp.dot(wh_ref[0], hcat, preferred_element_type=jnp.float32)
        p_scr[i] = q
        _accum_partial(q, sum_acc, sq_acc)

    if n_stages > 2:
        @pl.when(jnp.logical_and(s > 0, s < n_stages - 1))
        def _phase_mid():
            h = jnp.maximum(p_scr[i] * scale_scr[...] + shift_scr[...], 0.0)
            q = jnp.dot(wh_ref[s], h, preferred_element_type=jnp.float32)
            p_scr[i] = q
            _accum_partial(q, sum_acc, sq_acc)

    @pl.when(s == n_stages - 1)
    def _phase_out():
        h = jnp.maximum(p_scr[i] * scale_scr[...] + shift_scr[...], 0.0)
        out = jnp.dot(wl_ref[...], h, preferred_element_type=jnp.float32) + bl_ref[...]
        for g in range(group_b):
            o_ref[g] = out[:, g * l:(g + 1) * l].astype(o_ref.dtype)


def kernel(x, w1, b1, wh, bh, gamma, beta, wl, bl):
    n, cin, l = x.shape
    d = w1.shape[0]
    cout = wl.shape[0]
    n_hidden = wh.shape[0]
    n_stages = n_hidden + 1
    last = n_stages - 1

    group_b = next(g for g in (4, 2, 1) if n % g == 0)
    steps = n // group_b

    body = functools.partial(_fused_body, group_b=group_b, n_stages=n_stages,
                             l=l, m=float(n * l))

    x_spec = pl.BlockSpec(
        (group_b, cin, l),
        lambda s, i: (jnp.where(s == 0, i, steps - 1), 0, 0))
    o_spec = pl.BlockSpec(
        (group_b, cout, l),
        lambda s, i: (jnp.where(s == last, i, 0), 0, 0))

    def const(a):
        return pl.BlockSpec(a.shape, lambda *_: (0,) * a.ndim)

    return pl.pallas_call(
        body,
        grid=(n_stages, steps),
        in_specs=[x_spec, const(w1), const(b1), const(wh), const(bh),
                  const(gamma), const(beta), const(wl), const(bl)],
        out_specs=o_spec,
        out_shape=jax.ShapeDtypeStruct((n, cout, l), x.dtype),
        scratch_shapes=[
            pltpu.VMEM((steps, d, group_b * l), jnp.float32),
            pltpu.VMEM((d, 128), jnp.float32),
            pltpu.VMEM((d, 128), jnp.float32),
            pltpu.VMEM((d, 1), jnp.float32),
            pltpu.VMEM((d, 1), jnp.float32),
        ],
        compiler_params=pltpu.CompilerParams(
            dimension_semantics=("arbitrary", "arbitrary"),
            vmem_limit_bytes=60 * 1024 * 1024),
    )(x, w1, b1, wh, bh, gamma, beta, wl, bl)
